# baseline (speedup 1.0000x reference)
import functools

import jax
import jax.numpy as jnp
from jax import lax
from jax.experimental import pallas as pl
from jax.experimental.pallas import tpu as pltpu
from jax.experimental.pallas import tpu_sc as plsc

_NC = 2
_NS = 16
_L = 16

_CC = 512
_SEGE = 1024
_G = 64


def _sc_aggregate(entity, node_ids, src, dst, ety, *, N, D, R):
    f32 = jnp.float32
    i32 = jnp.int32
    RH = R // 2
    NPASS = -(-N // (2 * _CC))
    NPAD = NPASS * 2 * _CC
    ACCROWS = 2 * _CC + _NS
    HR = (_CC * R) // 128
    assert (_CC * R) % 128 == 0 and HR % 128 == 0

    E = src.shape[0]
    EP = E // _NS
    NSEG = EP // _SEGE
    assert EP % _SEGE == 0 and _SEGE % _L == 0
    assert (R & (R - 1)) == 0
    RB = R.bit_length() - 1
    assert NPASS <= _L

    mesh = plsc.VectorSubcoreMesh(core_axis_name="c", subcore_axis_name="s")

    def body(ent_h, nid_h, src_h, dst_h, ety_h, agg_o, hist_o,
             nid_v, srcb, dstb, etyb, acode, agid, stage_g, stage_s, idxst,
             rowbuf, histv, acc_s, hist_s, zsf, zspm, smo, smc,
             gsem0, gsem1, msem):
        gsem = (gsem0, gsem1)
        cid = lax.axis_index("c")
        sid = lax.axis_index("s")
        iota = lax.iota(i32, _L)
        zf = jnp.zeros((_L,), f32)
        zi = jnp.zeros((_L,), i32)

        pltpu.sync_copy(nid_h, nid_v)

        def _zero_rb(r, _):
            def inner(c, _):
                rowbuf[0, r, pl.ds(c * _L, _L)] = zf
                return 0
            return lax.fori_loop(0, D // _L, inner, 0)
        lax.fori_loop(0, 32, _zero_rb, 0)

        def _zero_hv(r, _):
            def inner(c, _):
                histv[r, pl.ds(c * _L, _L)] = zi
                return 0
            return lax.fori_loop(0, 128 // _L, inner, 0)
        lax.fori_loop(0, 32, _zero_hv, 0)

        @pl.when(sid < 4)
        def _():
            pltpu.sync_copy(rowbuf.at[0, pl.ds(0, 8)],
                            zsf.at[pl.ds(sid * 8, 8)])
            pltpu.sync_copy(histv.at[pl.ds(0, 8)], zspm.at[pl.ds(sid * 8, 8)])
        plsc.subcore_barrier()

        def cnt_seg(s, cnts):
            ebase = sid * EP + s * _SEGE
            pltpu.sync_copy(dst_h.at[pl.ds(ebase, _SEGE)], dstb)

            def cnt_vreg(i, cnts):
                d = dstb[pl.ds(i * _L, _L)]
                mine = (d >= 0) & (lax.rem(lax.div(d, _CC), 2) == cid)
                pidx = lax.div(d, 2 * _CC)
                for p in range(NPASS):
                    pc = plsc.all_reduce_population_count(mine & (pidx == p))
                    cnts = cnts + jnp.where(iota == p, pc, 0)
                return cnts

            return lax.fori_loop(0, _SEGE // _L, cnt_vreg, cnts)

        cnts = lax.fori_loop(0, NSEG, cnt_seg, zi)
        offv = plsc.cumsum(cnts) - cnts
        for p in range(NPASS):
            smo[p] = offv[p]
            smc[p] = cnts[p]

        def fill_seg(s, offc):
            ebase = sid * EP + s * _SEGE
            d1 = pltpu.async_copy(src_h.at[pl.ds(ebase, _SEGE)], srcb, msem)
            d2 = pltpu.async_copy(dst_h.at[pl.ds(ebase, _SEGE)], dstb, msem)
            d3 = pltpu.async_copy(ety_h.at[pl.ds(ebase, _SEGE)], etyb, msem)
            d1.wait()
            d2.wait()
            d3.wait()

            def fill_vreg(i, offc):
                d = dstb[pl.ds(i * _L, _L)]
                t = etyb[pl.ds(i * _L, _L)]
                sv = srcb[pl.ds(i * _L, _L)]
                gi = plsc.load_gather(nid_v, [sv])
                code = d * R + t
                mine = (d >= 0) & (lax.rem(lax.div(d, _CC), 2) == cid)
                pidx = lax.div(d, 2 * _CC)
                for p in range(NPASS):
                    m = mine & (pidx == p)
                    ob = offc[p]
                    plsc.store_compressed(acode.at[pl.ds(ob, _L)], code, mask=m)
                    plsc.store_compressed(agid.at[pl.ds(ob, _L)], gi, mask=m)
                    pc = plsc.all_reduce_population_count(m)
                    offc = offc + jnp.where(iota == p, pc, 0)
                return offc

            return lax.fori_loop(0, _SEGE // _L, fill_vreg, offc)

        lax.fori_loop(0, NSEG, fill_seg, offv)

        def pass_body(p, _):
            lo = p * (2 * _CC) + cid * _CC

            r0 = sid * (2 * _CC // _NS)
            for k in range(2 * _CC // _NS // 32):
                pltpu.sync_copy(zsf, acc_s.at[pl.ds(r0 + k * 32, 32)])

            @pl.when(sid == 0)
            def _():
                pltpu.sync_copy(zsf.at[pl.ds(0, 16)],
                                acc_s.at[pl.ds(2 * _CC, 16)])
            pltpu.sync_copy(zspm.at[pl.ds(0, HR // _NS)],
                            hist_s.at[pl.ds(sid * (HR // _NS), HR // _NS)])
            for k in range(HR // 32):
                pltpu.sync_copy(zspm, histv.at[pl.ds(k * 32, 32)])
            plsc.subcore_barrier()

            boff = smo[p]
            bcnt = smc[p]
            nch = (bcnt + _G - 1) // _G

            def fire(c, k):
                @pl.when(c < nch)
                def _():
                    b = boff + c * _G
                    for j in range(_G // _L):
                        lane = c * _G + j * _L + iota
                        m = lane < bcnt
                        code16 = acode[pl.ds(b + j * _L, _L)]
                        gi16 = agid[pl.ds(b + j * _L, _L)]
                        d16 = lax.shift_right_logical(code16, RB)
                        t16 = lax.bitwise_and(code16, R - 1)
                        srow = (d16 - lo) + jnp.where(t16 >= RH, _CC, 0)
                        hq = jnp.where(m, code16 - lo * R, 0)
                        cnt16, lastm = plsc.scan_count(hq, mask=m)
                        plsc.addupdate_scatter(
                            histv,
                            [lax.shift_right_logical(hq, 7),
                             lax.bitwise_and(hq, 127)],
                            cnt16, mask=lastm)
                        stage_g[k, pl.ds(j * _L, _L)] = jnp.where(m, gi16, 0)
                        stage_s[k, pl.ds(j * _L, _L)] = (
                            jnp.where(m, srow, 2 * _CC + sid))
                    pltpu.async_copy(ent_h.at[stage_g.at[k]],
                                     rowbuf.at[k], gsem[k])

            def scat(c, k):
                @pl.when(c < nch)
                def _():
                    pltpu.make_async_copy(ent_h.at[stage_g.at[k]],
                                          rowbuf.at[k], gsem[k]).wait()
                    pltpu.sync_copy(rowbuf.at[k], acc_s.at[stage_s.at[k]],
                                    add=True)

            fire(jnp.int32(0), 0)

            def drain2(i, _):
                cc = i * 2
                fire(cc + 1, 1)
                scat(cc, 0)
                fire(cc + 2, 0)
                scat(cc + 1, 1)
                return 0

            lax.fori_loop(0, (nch + 1) // 2, drain2, 0)

            def flush(c, _):
                for k in range(128 // _L):
                    idxst[pl.ds(k * _L, _L)] = c * 128 + k * _L + iota
                pltpu.sync_copy(histv.at[pl.ds(c * 128, 128)],
                                hist_s.at[idxst], add=True)
                return 0

            lax.fori_loop(0, HR // 128, flush, 0)
            plsc.subcore_barrier()

            rpt = _CC // _NS
            n0 = lo + sid * rpt
            pltpu.sync_copy(acc_s.at[pl.ds(sid * rpt, rpt)],
                            agg_o.at[pl.ds(n0, rpt)])
            pltpu.sync_copy(acc_s.at[pl.ds(_CC + sid * rpt, rpt)],
                            agg_o.at[pl.ds(NPAD + n0, rpt)])
            q = p * 2 + cid
            hpt = HR // _NS
            pltpu.sync_copy(hist_s.at[pl.ds(sid * hpt, hpt)],
                            hist_o.at[pl.ds(q * HR + sid * hpt, hpt)])
            plsc.subcore_barrier()
            return 0

        lax.fori_loop(0, NPASS, pass_body, 0)

    agg, hist = pl.kernel(
        body,
        out_type=[
            jax.ShapeDtypeStruct((2 * NPAD, D), f32),
            jax.ShapeDtypeStruct((NPASS * 2 * HR, 128), i32),
        ],
        mesh=mesh,
        compiler_params=pltpu.CompilerParams(
            needs_layout_passes=False, use_tc_tiling_on_sc=False),
        scratch_types=[
            pltpu.VMEM((N,), i32),
            pltpu.VMEM((_SEGE,), i32),
            pltpu.VMEM((_SEGE,), i32),
            pltpu.VMEM((_SEGE,), i32),
            pltpu.VMEM((EP + 64,), i32),
            pltpu.VMEM((EP + 64,), i32),
            pltpu.VMEM((2, _G), i32),
            pltpu.VMEM((2, _G), i32),
            pltpu.VMEM((128,), i32),
            pltpu.VMEM((2, _G, D), f32),
            pltpu.VMEM((HR, 128), i32),
            pltpu.VMEM_SHARED((ACCROWS, D), f32),
            pltpu.VMEM_SHARED((HR, 128), i32),
            pltpu.VMEM_SHARED((32, D), f32),
            pltpu.VMEM_SHARED((32, 128), i32),
            pltpu.SMEM((_L,), i32),
            pltpu.SMEM((_L,), i32),
            pltpu.SemaphoreType.DMA,
            pltpu.SemaphoreType.DMA,
            pltpu.SemaphoreType.DMA,
        ],
    )(entity, node_ids, src, dst, ety)
    return agg, hist, NPAD


def _tc_body(RH, hist_ref, aggI_ref, aggO_ref, relw_ref, wI_ref, wO_ref,
             bI_ref, bO_ref, fcw_ref, fcb_ref, out_ref):
    f32 = jnp.float32
    hist = hist_ref[...].astype(f32)
    histI = hist[:, :RH]
    histO = hist[:, RH:]
    relw = relw_ref[...]
    relI = jnp.dot(histI, relw[:RH, :], preferred_element_type=f32)
    relO = jnp.dot(histO, relw[RH:, :], preferred_element_type=f32)
    degI = jnp.sum(histI, axis=1, keepdims=True)
    degO = jnp.sum(histO, axis=1, keepdims=True)
    xI = aggI_ref[...] - relI
    xO = aggO_ref[...] - relO
    dn = (((1,), (1,)), ((), ()))
    sI = lax.dot_general(xI, wI_ref[...], dn, preferred_element_type=f32)
    sO = lax.dot_general(xO, wO_ref[...], dn, preferred_element_type=f32)
    summed = sI + sO + degI * bI_ref[...] + degO * bO_ref[...]
    nout = summed / jnp.maximum(degI + degO, 1.0)
    logits = lax.dot_general(nout, fcw_ref[...], dn, preferred_element_type=f32)
    out_ref[...] = jax.nn.sigmoid(logits + fcb_ref[...])


def kernel(node_ids, edge_index, etype, entity, rel_embeds,
           W_I_w, W_I_b, W_O_w, W_O_b, fc_w, fc_b):
    N, D = entity.shape
    R = rel_embeds.shape[0]
    T = fc_w.shape[0]
    E = etype.shape[0]
    i32 = jnp.int32
    f32 = jnp.float32

    node_ids = node_ids.astype(i32)
    src = edge_index[0].astype(i32)
    dst = edge_index[1].astype(i32)
    ety = etype.astype(i32)

    EPT = -(-E // (_NS * _SEGE)) * (_NS * _SEGE)
    if EPT != E:
        pad = EPT - E
        src = jnp.concatenate([src, jnp.zeros((pad,), i32)])
        dst = jnp.concatenate([dst, jnp.full((pad,), -1, i32)])
        ety = jnp.concatenate([ety, jnp.zeros((pad,), i32)])

    agg, hist, NPAD = _sc_aggregate(entity, node_ids, src, dst, ety,
                                    N=N, D=D, R=R)
    aggI = agg[:N]
    aggO = agg[NPAD:NPAD + N]
    NPASS = NPAD // (2 * _CC)
    HR = (_CC * R) // 128
    hist_n = hist.reshape(NPASS * 2 * _CC, R)[:N]

    BN = 1000
    assert N % BN == 0
    grid = N // BN
    out = pl.pallas_call(
        functools.partial(_tc_body, R // 2),
        grid=(grid,),
        in_specs=[
            pl.BlockSpec((BN, R), lambda i: (i, 0)),
            pl.BlockSpec((BN, D), lambda i: (i, 0)),
            pl.BlockSpec((BN, D), lambda i: (i, 0)),
            pl.BlockSpec((R, D), lambda i: (0, 0)),
            pl.BlockSpec((D, D), lambda i: (0, 0)),
            pl.BlockSpec((D, D), lambda i: (0, 0)),
            pl.BlockSpec((1, D), lambda i: (0, 0)),
            pl.BlockSpec((1, D), lambda i: (0, 0)),
            pl.BlockSpec((T, D), lambda i: (0, 0)),
            pl.BlockSpec((1, T), lambda i: (0, 0)),
        ],
        out_specs=pl.BlockSpec((BN, T), lambda i: (i, 0)),
        out_shape=jax.ShapeDtypeStruct((N, T), f32),
    )(hist_n, aggI, aggO, rel_embeds, W_I_w, W_O_w,
      W_I_b.reshape(1, D), W_O_b.reshape(1, D), fc_w, fc_b.reshape(1, T))
    return out


# speedup vs baseline: 1.1843x; 1.1843x over previous
import functools

import jax
import jax.numpy as jnp
from jax import lax
from jax.experimental import pallas as pl
from jax.experimental.pallas import tpu as pltpu
from jax.experimental.pallas import tpu_sc as plsc

_NC = 2
_NS = 16
_L = 16

_CC = 768
_SEGE = 1024
_G = 32


def _sc_aggregate(entity, node_ids, src, dst, ety, *, N, D, R):
    f32 = jnp.float32
    i32 = jnp.int32
    RH = R // 2
    NPASS = -(-N // (2 * _CC))
    NPAD = NPASS * 2 * _CC
    ACCROWS = 2 * _CC + _NS
    HR = (_CC * R) // 128
    assert (_CC * R) % 128 == 0 and HR % 128 == 0

    E = src.shape[0]
    EP = E // _NS
    NSEG = EP // _SEGE
    assert EP % _SEGE == 0 and _SEGE % _L == 0
    assert (R & (R - 1)) == 0
    RB = R.bit_length() - 1
    assert NPASS <= _L

    mesh = plsc.VectorSubcoreMesh(core_axis_name="c", subcore_axis_name="s")

    def body(ent_h, nid_h, src_h, dst_h, ety_h, agg_o, hist_o,
             nid_v, srcb, dstb, etyb, acode, agid, stage_g, stage_s, idxst,
             rowbuf, histv, acc_s, hist_s, zsf, zspm, smo, smc,
             gsem0, gsem1, msem):
        gsem = (gsem0, gsem1)
        cid = lax.axis_index("c")
        sid = lax.axis_index("s")
        iota = lax.iota(i32, _L)
        zf = jnp.zeros((_L,), f32)
        zi = jnp.zeros((_L,), i32)

        pltpu.sync_copy(nid_h, nid_v)

        def _zero_rb(r, _):
            def inner(c, _):
                rowbuf[0, r, pl.ds(c * _L, _L)] = zf
                return 0
            return lax.fori_loop(0, D // _L, inner, 0)
        lax.fori_loop(0, 32, _zero_rb, 0)

        def _zero_hv(r, _):
            def inner(c, _):
                histv[r, pl.ds(c * _L, _L)] = zi
                return 0
            return lax.fori_loop(0, 128 // _L, inner, 0)
        lax.fori_loop(0, 32, _zero_hv, 0)

        @pl.when(sid < 4)
        def _():
            pltpu.sync_copy(rowbuf.at[0, pl.ds(0, 8)],
                            zsf.at[pl.ds(sid * 8, 8)])
        pltpu.sync_copy(histv.at[pl.ds(0, 8)], zspm.at[pl.ds(sid * 8, 8)])
        plsc.subcore_barrier()

        def cnt_seg(s, cnts):
            ebase = sid * EP + s * _SEGE
            pltpu.sync_copy(dst_h.at[pl.ds(ebase, _SEGE)], dstb)

            def cnt_vreg(i, cnts):
                d = dstb[pl.ds(i * _L, _L)]
                mine = (d >= 0) & (lax.rem(lax.div(d, _CC), 2) == cid)
                pidx = lax.div(d, 2 * _CC)
                for p in range(NPASS):
                    pc = plsc.all_reduce_population_count(mine & (pidx == p))
                    cnts = cnts + jnp.where(iota == p, pc, 0)
                return cnts

            return lax.fori_loop(0, _SEGE // _L, cnt_vreg, cnts)

        cnts = lax.fori_loop(0, NSEG, cnt_seg, zi)
        offv = plsc.cumsum(cnts) - cnts
        for p in range(NPASS):
            smo[p] = offv[p]
            smc[p] = cnts[p]

        def fill_seg(s, offc):
            ebase = sid * EP + s * _SEGE
            d1 = pltpu.async_copy(src_h.at[pl.ds(ebase, _SEGE)], srcb, msem)
            d2 = pltpu.async_copy(dst_h.at[pl.ds(ebase, _SEGE)], dstb, msem)
            d3 = pltpu.async_copy(ety_h.at[pl.ds(ebase, _SEGE)], etyb, msem)
            d1.wait()
            d2.wait()
            d3.wait()

            def fill_vreg(i, offc):
                d = dstb[pl.ds(i * _L, _L)]
                t = etyb[pl.ds(i * _L, _L)]
                sv = srcb[pl.ds(i * _L, _L)]
                gi = plsc.load_gather(nid_v, [sv])
                code = d * R + t
                mine = (d >= 0) & (lax.rem(lax.div(d, _CC), 2) == cid)
                pidx = lax.div(d, 2 * _CC)
                for p in range(NPASS):
                    m = mine & (pidx == p)
                    ob = offc[p]
                    plsc.store_compressed(acode.at[pl.ds(ob, _L)], code, mask=m)
                    plsc.store_compressed(agid.at[pl.ds(ob, _L)], gi, mask=m)
                    pc = plsc.all_reduce_population_count(m)
                    offc = offc + jnp.where(iota == p, pc, 0)
                return offc

            return lax.fori_loop(0, _SEGE // _L, fill_vreg, offc)

        lax.fori_loop(0, NSEG, fill_seg, offv)

        def pass_body(p, _):
            lo = p * (2 * _CC) + cid * _CC

            r0 = sid * (2 * _CC // _NS)
            for k in range(2 * _CC // _NS // 32):
                pltpu.sync_copy(zsf, acc_s.at[pl.ds(r0 + k * 32, 32)])

            @pl.when(sid == 0)
            def _():
                pltpu.sync_copy(zsf.at[pl.ds(0, 16)],
                                acc_s.at[pl.ds(2 * _CC, 16)])
            pltpu.sync_copy(zspm.at[pl.ds(0, HR // _NS)],
                            hist_s.at[pl.ds(sid * (HR // _NS), HR // _NS)])
            for k in range(HR // 128):
                pltpu.sync_copy(zspm, histv.at[pl.ds(k * 128, 128)])
            plsc.subcore_barrier()

            boff = smo[p]
            bcnt = smc[p]
            nch = (bcnt + _G - 1) // _G

            def fire(c, k):
                @pl.when(c < nch)
                def _():
                    b = boff + c * _G
                    for j in range(_G // _L):
                        lane = c * _G + j * _L + iota
                        m = lane < bcnt
                        code16 = acode[pl.ds(b + j * _L, _L)]
                        gi16 = agid[pl.ds(b + j * _L, _L)]
                        d16 = lax.shift_right_logical(code16, RB)
                        t16 = lax.bitwise_and(code16, R - 1)
                        srow = (d16 - lo) + jnp.where(t16 >= RH, _CC, 0)
                        hq = jnp.where(m, code16 - lo * R, 0)
                        cnt16, lastm = plsc.scan_count(hq, mask=m)
                        plsc.addupdate_scatter(
                            histv,
                            [lax.shift_right_logical(hq, 7),
                             lax.bitwise_and(hq, 127)],
                            cnt16, mask=lastm)
                        stage_g[k, pl.ds(j * _L, _L)] = jnp.where(m, gi16, 0)
                        stage_s[k, pl.ds(j * _L, _L)] = (
                            jnp.where(m, srow, 2 * _CC + sid))
                    pltpu.async_copy(ent_h.at[stage_g.at[k]],
                                     rowbuf.at[k], gsem[k])

            def scat(c, k):
                @pl.when(c < nch)
                def _():
                    pltpu.make_async_copy(ent_h.at[stage_g.at[k]],
                                          rowbuf.at[k], gsem[k]).wait()
                    pltpu.sync_copy(rowbuf.at[k], acc_s.at[stage_s.at[k]],
                                    add=True)

            fire(jnp.int32(0), 0)

            def drain2(i, _):
                cc = i * 2
                fire(cc + 1, 1)
                scat(cc, 0)
                fire(cc + 2, 0)
                scat(cc + 1, 1)
                return 0

            lax.fori_loop(0, (nch + 1) // 2, drain2, 0)

            def flush(c, _):
                for k in range(128 // _L):
                    idxst[pl.ds(k * _L, _L)] = c * 128 + k * _L + iota
                pltpu.sync_copy(histv.at[pl.ds(c * 128, 128)],
                                hist_s.at[idxst], add=True)
                return 0

            lax.fori_loop(0, HR // 128, flush, 0)
            plsc.subcore_barrier()

            rpt = _CC // _NS
            n0 = lo + sid * rpt
            pltpu.sync_copy(acc_s.at[pl.ds(sid * rpt, rpt)],
                            agg_o.at[pl.ds(n0, rpt)])
            pltpu.sync_copy(acc_s.at[pl.ds(_CC + sid * rpt, rpt)],
                            agg_o.at[pl.ds(NPAD + n0, rpt)])
            q = p * 2 + cid
            hpt = HR // _NS
            pltpu.sync_copy(hist_s.at[pl.ds(sid * hpt, hpt)],
                            hist_o.at[pl.ds(q * HR + sid * hpt, hpt)])
            plsc.subcore_barrier()
            return 0

        lax.fori_loop(0, NPASS, pass_body, 0)

    agg, hist = pl.kernel(
        body,
        out_type=[
            jax.ShapeDtypeStruct((2 * NPAD, D), f32),
            jax.ShapeDtypeStruct((NPASS * 2 * HR, 128), i32),
        ],
        mesh=mesh,
        compiler_params=pltpu.CompilerParams(
            needs_layout_passes=False, use_tc_tiling_on_sc=False),
        scratch_types=[
            pltpu.VMEM((N,), i32),
            pltpu.VMEM((_SEGE,), i32),
            pltpu.VMEM((_SEGE,), i32),
            pltpu.VMEM((_SEGE,), i32),
            pltpu.VMEM((EP + 64,), i32),
            pltpu.VMEM((EP + 64,), i32),
            pltpu.VMEM((2, _G), i32),
            pltpu.VMEM((2, _G), i32),
            pltpu.VMEM((128,), i32),
            pltpu.VMEM((2, _G, D), f32),
            pltpu.VMEM((HR, 128), i32),
            pltpu.VMEM_SHARED((ACCROWS, D), f32),
            pltpu.VMEM_SHARED((HR, 128), i32),
            pltpu.VMEM_SHARED((32, D), f32),
            pltpu.VMEM_SHARED((128, 128), i32),
            pltpu.SMEM((_L,), i32),
            pltpu.SMEM((_L,), i32),
            pltpu.SemaphoreType.DMA,
            pltpu.SemaphoreType.DMA,
            pltpu.SemaphoreType.DMA,
        ],
    )(entity, node_ids, src, dst, ety)
    return agg, hist, NPAD


def _tc_body(RH, hist_ref, aggI_ref, aggO_ref, relw_ref, wI_ref, wO_ref,
             bI_ref, bO_ref, fcw_ref, fcb_ref, out_ref):
    f32 = jnp.float32
    hist = hist_ref[...].astype(f32)
    histI = hist[:, :RH]
    histO = hist[:, RH:]
    relw = relw_ref[...]
    relI = jnp.dot(histI, relw[:RH, :], preferred_element_type=f32)
    relO = jnp.dot(histO, relw[RH:, :], preferred_element_type=f32)
    degI = jnp.sum(histI, axis=1, keepdims=True)
    degO = jnp.sum(histO, axis=1, keepdims=True)
    xI = aggI_ref[...] - relI
    xO = aggO_ref[...] - relO
    dn = (((1,), (1,)), ((), ()))
    sI = lax.dot_general(xI, wI_ref[...], dn, preferred_element_type=f32)
    sO = lax.dot_general(xO, wO_ref[...], dn, preferred_element_type=f32)
    summed = sI + sO + degI * bI_ref[...] + degO * bO_ref[...]
    nout = summed / jnp.maximum(degI + degO, 1.0)
    logits = lax.dot_general(nout, fcw_ref[...], dn, preferred_element_type=f32)
    out_ref[...] = jax.nn.sigmoid(logits + fcb_ref[...])


def kernel(node_ids, edge_index, etype, entity, rel_embeds,
           W_I_w, W_I_b, W_O_w, W_O_b, fc_w, fc_b):
    N, D = entity.shape
    R = rel_embeds.shape[0]
    T = fc_w.shape[0]
    E = etype.shape[0]
    i32 = jnp.int32
    f32 = jnp.float32

    node_ids = node_ids.astype(i32)
    src = edge_index[0].astype(i32)
    dst = edge_index[1].astype(i32)
    ety = etype.astype(i32)

    EPT = -(-E // (_NS * _SEGE)) * (_NS * _SEGE)
    if EPT != E:
        pad = EPT - E
        src = jnp.concatenate([src, jnp.zeros((pad,), i32)])
        dst = jnp.concatenate([dst, jnp.full((pad,), -1, i32)])
        ety = jnp.concatenate([ety, jnp.zeros((pad,), i32)])

    agg, hist, NPAD = _sc_aggregate(entity, node_ids, src, dst, ety,
                                    N=N, D=D, R=R)
    aggI = agg[:N]
    aggO = agg[NPAD:NPAD + N]
    NPASS = NPAD // (2 * _CC)
    HR = (_CC * R) // 128
    hist_n = hist.reshape(NPASS * 2 * _CC, R)[:N]

    BN = 1000
    assert N % BN == 0
    grid = N // BN
    out = pl.pallas_call(
        functools.partial(_tc_body, R // 2),
        grid=(grid,),
        in_specs=[
            pl.BlockSpec((BN, R), lambda i: (i, 0)),
            pl.BlockSpec((BN, D), lambda i: (i, 0)),
            pl.BlockSpec((BN, D), lambda i: (i, 0)),
            pl.BlockSpec((R, D), lambda i: (0, 0)),
            pl.BlockSpec((D, D), lambda i: (0, 0)),
            pl.BlockSpec((D, D), lambda i: (0, 0)),
            pl.BlockSpec((1, D), lambda i: (0, 0)),
            pl.BlockSpec((1, D), lambda i: (0, 0)),
            pl.BlockSpec((T, D), lambda i: (0, 0)),
            pl.BlockSpec((1, T), lambda i: (0, 0)),
        ],
        out_specs=pl.BlockSpec((BN, T), lambda i: (i, 0)),
        out_shape=jax.ShapeDtypeStruct((N, T), f32),
    )(hist_n, aggI, aggO, rel_embeds, W_I_w, W_O_w,
      W_I_b.reshape(1, D), W_O_b.reshape(1, D), fc_w, fc_b.reshape(1, T))
    return out
